# baseline (speedup 1.0000x reference)
import math

import jax
import jax.numpy as jnp
from jax.experimental import pallas as pl

NEG_LARGE = -1e30
RATIO = 0.8


def _sage(h, src, dst, valid, lw, lb, uw):
    n = h.shape[0]
    t = jax.nn.relu(h @ lw.T + lb)
    msg = jnp.where(valid[:, None], t[src], NEG_LARGE)
    aggr = jax.ops.segment_max(msg, dst, num_segments=n)
    aggr = jnp.maximum(aggr, t)
    return jax.nn.relu(jnp.concatenate([aggr, h], axis=1) @ uw.T)


def _pool(h, src, dst, valid, w):
    n = h.shape[0]
    k = int(math.ceil(RATIO * n))
    s = (h @ w) / jnp.linalg.norm(w)
    _, perm = jax.lax.top_k(s, k)
    sel = jnp.sort(perm)
    vals = jnp.tanh(s[sel])
    h_new = h[sel] * vals[:, None]
    new_idx = jnp.full((n,), -1, dtype=src.dtype).at[sel].set(
        jnp.arange(k, dtype=src.dtype))
    ns = new_idx[src]
    nd = new_idx[dst]
    v = valid & (ns >= 0) & (nd >= 0)
    return h_new, jnp.where(v, ns, 0), jnp.where(v, nd, 0), v


def _readout(h):
    return jnp.concatenate(
        [jnp.max(h, axis=0, keepdims=True), jnp.mean(h, axis=0, keepdims=True)],
        axis=1)


def _mlp_body(embed_ref, w1_ref, b1_ref, w2_ref, b2_ref, w3_ref, b3_ref,
              out_ref, emb_ref):
    e = embed_ref[...]
    t = jnp.maximum(
        jnp.dot(e, w1_ref[...].T, preferred_element_type=jnp.float32)
        + b1_ref[...], 0.0)
    t = jnp.maximum(
        jnp.dot(t, w2_ref[...].T, preferred_element_type=jnp.float32)
        + b2_ref[...], 0.0)
    o = jax.nn.sigmoid(
        jnp.dot(t, w3_ref[...].T, preferred_element_type=jnp.float32)
        + b3_ref[...])
    out_ref[...] = o
    emb_ref[...] = e


def _mlp(embed, lin1_w, lin1_b, lin2_w, lin2_b, lin3_w, lin3_b):
    w3p = jnp.pad(lin3_w, ((0, 7), (0, 0)))
    b3p = jnp.pad(lin3_b.reshape(1, 1), ((0, 0), (0, 7)))
    out2, emb2 = pl.pallas_call(
        _mlp_body,
        out_shape=(
            jax.ShapeDtypeStruct((1, 8), jnp.float32),
            jax.ShapeDtypeStruct((1, 64), jnp.float32),
        ),
    )(embed, lin1_w, lin1_b.reshape(1, -1), lin2_w, lin2_b.reshape(1, -1),
      w3p, b3p)
    return out2[:1, 0], emb2[0]


def kernel(x, edge_index, batch, emb, c1_lin_w, c1_lin_b, c1_upd_w, p1_w,
           c2_lin_w, c2_lin_b, c2_upd_w, p2_w, c3_lin_w, c3_lin_b, c3_upd_w,
           p3_w, lin1_w, lin1_b, lin2_w, lin2_b, lin3_w, lin3_b):
    h = emb[x[:, 0]]
    src = edge_index[0]
    dst = edge_index[1]
    valid = jnp.ones((src.shape[0],), dtype=bool)

    h = _sage(h, src, dst, valid, c1_lin_w, c1_lin_b, c1_upd_w)
    h, src, dst, valid = _pool(h, src, dst, valid, p1_w)
    x1 = _readout(h)
    h = _sage(h, src, dst, valid, c2_lin_w, c2_lin_b, c2_upd_w)
    h, src, dst, valid = _pool(h, src, dst, valid, p2_w)
    x2 = _readout(h)
    h = _sage(h, src, dst, valid, c3_lin_w, c3_lin_b, c3_upd_w)
    h, src, dst, valid = _pool(h, src, dst, valid, p3_w)
    x3 = _readout(h)

    embed = x1 + x2 + x3
    return _mlp(embed, lin1_w, lin1_b, lin2_w, lin2_b, lin3_w, lin3_b)


# speedup vs baseline: 3.5741x; 3.5741x over previous
import functools
import math

import jax
import jax.numpy as jnp
from jax import lax
from jax.experimental import pallas as pl
from jax.experimental.pallas import tpu as pltpu
from jax.experimental.pallas import tpu_sc as plsc

NEG_LARGE = -1e30
RATIO = 0.8
N_EDGES = 1600000
NW = 32
PER_W = N_EDGES // NW
REMAP_W = 2000


@functools.cache
def _make_remap(n):
    mesh = plsc.VectorSubcoreMesh(core_axis_name="c", subcore_axis_name="s")

    @functools.partial(
        pl.kernel,
        out_type=(jax.ShapeDtypeStruct((N_EDGES,), jnp.int32),
                  jax.ShapeDtypeStruct((N_EDGES,), jnp.int32)),
        mesh=mesh,
        compiler_params=pltpu.CompilerParams(needs_layout_passes=False),
        scratch_types=[
            pltpu.VMEM((n,), jnp.int32),
            pltpu.VMEM((REMAP_W,), jnp.int32),
            pltpu.VMEM((REMAP_W,), jnp.int32),
            pltpu.VMEM((REMAP_W,), jnp.int32),
            pltpu.VMEM((REMAP_W,), jnp.int32),
        ],
    )
    def k(src_hbm, dst_hbm, tbl_hbm, ns_hbm, nd_hbm, tbl_v, s_v, d_v,
          os_v, od_v):
        wid = lax.axis_index("s") * 2 + lax.axis_index("c")
        pltpu.sync_copy(tbl_hbm, tbl_v)
        base = wid * PER_W

        def body(w, carry):
            off = base + w * REMAP_W
            pltpu.sync_copy(src_hbm.at[pl.ds(off, REMAP_W)], s_v)
            pltpu.sync_copy(dst_hbm.at[pl.ds(off, REMAP_W)], d_v)

            def inner(i, c):
                sl = pl.ds(i * 16, 16)
                si = s_v[sl]
                di = d_v[sl]
                gs = plsc.load_gather(tbl_v, [jnp.maximum(si, 0)])
                gd = plsc.load_gather(tbl_v, [jnp.maximum(di, 0)])
                ok = (si >= 0) & (di >= 0) & (gs >= 0) & (gd >= 0)
                neg = jnp.full((16,), -1, jnp.int32)
                os_v[sl] = jnp.where(ok, gs, neg)
                od_v[sl] = jnp.where(ok, gd, neg)
                return c

            lax.fori_loop(0, REMAP_W // 16, inner, 0, unroll=4)
            pltpu.sync_copy(os_v, ns_hbm.at[pl.ds(off, REMAP_W)])
            pltpu.sync_copy(od_v, nd_hbm.at[pl.ds(off, REMAP_W)])
            return carry

        lax.fori_loop(0, PER_W // REMAP_W, body, 0)

    return k


SEG_W = 2000
CHUNK = 128
STAGE = 160


@functools.cache
def _make_segmax(n):
    sz = n // NW
    sz4 = sz // 4
    mesh = plsc.VectorSubcoreMesh(core_axis_name="c", subcore_axis_name="s")

    @functools.partial(
        pl.kernel,
        out_type=jax.ShapeDtypeStruct((n // 4, 128), jnp.float32),
        mesh=mesh,
        compiler_params=pltpu.CompilerParams(needs_layout_passes=False),
        scratch_types=[
            pltpu.VMEM((sz4 + 1, 128), jnp.float32),
            pltpu.VMEM((SEG_W,), jnp.int32),
            pltpu.VMEM((SEG_W,), jnp.int32),
            pltpu.VMEM((STAGE,), jnp.int32),
            pltpu.VMEM((STAGE,), jnp.int32),
            pltpu.VMEM((STAGE,), jnp.int32),
            pltpu.VMEM((CHUNK, 128), jnp.float32),
            pltpu.VMEM((16, 128), jnp.float32),
            pltpu.SemaphoreType.DMA,
        ],
    )
    def k(t_hbm, src_hbm, dst_hbm, aggr_hbm, slab, s_v, d_v, st_q, st_m,
          st_d, rows, rows16, sem):
        wid = lax.axis_index("s") * 2 + lax.axis_index("c")
        lo = wid * sz
        q0 = wid * sz4
        pltpu.sync_copy(t_hbm.at[pl.ds(q0, sz4)], slab.at[pl.ds(0, sz4)])

        iota = lax.iota(jnp.int32, 16)
        rots = [(iota + s) & 15 for s in (1, 2, 4, 8)]
        rm1 = (iota + 15) & 15

        def _rot(v, r):
            return lax.gather(
                v, r[:, None],
                lax.GatherDimensionNumbers(
                    offset_dims=(), collapsed_slice_dims=(0,),
                    start_index_map=(0,)),
                slice_sizes=(1,),
                mode=lax.GatherScatterMode.PROMISE_IN_BOUNDS)

        def update_16(rows_ref, row_off, st_off):
            dvec = st_d[pl.ds(st_off, 16)]
            mvec = st_m[pl.ds(st_off, 16)]
            sk, sl = plsc.sort_key_val(dvec, iota)
            masks = [sk == _rot(sk, r) for r in rots]
            first = (sk != _rot(sk, rm1)) | (iota == 0)
            row = row_off + sl
            scol = _rot(mvec, sl) * 32
            skq = lax.shift_right_arithmetic(sk, 2)
            skc = (sk & 3) * 32

            def col(ci, c2):
                cc = jnp.full((16,), ci, jnp.int32)
                v = plsc.load_gather(rows_ref, [row, scol + cc])
                for m, r in zip(masks, rots):
                    v = jnp.where(m, jnp.maximum(v, _rot(v, r)), v)
                cur = plsc.load_gather(slab, [skq, skc + cc])
                v = jnp.maximum(v, cur)
                plsc.store_scatter(slab, [skq, skc + cc], v, mask=first)
                return c2

            lax.fori_loop(0, 32, col, 0, unroll=4)

        def fire(c):
            pltpu.async_copy(t_hbm.at[st_q.at[pl.ds(0, CHUNK)]], rows,
                             sem).wait()

            def fgroup(g, c2):
                update_16(rows, g * 16, g * 16)
                return c2

            lax.fori_loop(0, CHUNK // 16, fgroup, 0)
            st_q[pl.ds(0, 16)] = st_q[pl.ds(CHUNK, 16)]
            st_m[pl.ds(0, 16)] = st_m[pl.ds(CHUNK, 16)]
            st_d[pl.ds(0, 16)] = st_d[pl.ds(CHUNK, 16)]
            return c - CHUNK

        def window(w, c):
            off = w * SEG_W
            pltpu.sync_copy(src_hbm.at[pl.ds(off, SEG_W)], s_v)
            pltpu.sync_copy(dst_hbm.at[pl.ds(off, SEG_W)], d_v)

            def group(i, c1):
                sl = pl.ds(i * 16, 16)
                dvec = d_v[sl]
                svec = s_v[sl]
                m = (dvec >= lo) & (dvec < lo + sz)
                plsc.store_compressed(
                    st_q.at[pl.ds(c1, 16)],
                    lax.shift_right_arithmetic(svec, 2), mask=m)
                plsc.store_compressed(st_m.at[pl.ds(c1, 16)], svec & 3,
                                      mask=m)
                plsc.store_compressed(st_d.at[pl.ds(c1, 16)], dvec - lo,
                                      mask=m)
                c1 = c1 + jnp.sum(m.astype(jnp.int32))
                return lax.cond(c1 >= CHUNK, fire, lambda c2: c2, c1)

            return lax.fori_loop(0, SEG_W // 16, group, c)

        c = lax.fori_loop(0, N_EDGES // SEG_W, window, jnp.int32(0))

        def drain(g, c3):
            base = g * 16
            lane = base + lax.iota(jnp.int32, 16)
            ok = lane < c3
            idx = jnp.where(ok, st_q[pl.ds(base, 16)], 0)
            mpad = jnp.where(ok, st_m[pl.ds(base, 16)], 0)
            dloc = jnp.where(ok, st_d[pl.ds(base, 16)], sz)
            st_m[pl.ds(base, 16)] = mpad
            st_d[pl.ds(base, 16)] = dloc
            pltpu.async_copy(t_hbm.at[idx], rows16, sem).wait()
            update_16(rows16, 0, base)
            return c3

        lax.fori_loop(0, (c + 15) // 16, drain, c)
        pltpu.sync_copy(slab.at[pl.ds(0, sz4)], aggr_hbm.at[pl.ds(q0, sz4)])

    return k


def _sage(h, src, dst, lw, lb, uw):
    n = h.shape[0]
    t = jax.nn.relu(h @ lw.T + lb)
    n_pad = ((n + 1023) // 1024) * 1024
    if n_pad != n:
        t = jnp.pad(t, ((0, n_pad - n), (0, 0)))
    t4 = t.reshape(n_pad // 4, 128)
    aggr = _make_segmax(n_pad)(t4, src, dst).reshape(n_pad, 32)[:n]
    return jax.nn.relu(jnp.concatenate([aggr, h], axis=1) @ uw.T)


def _pool(h, src, dst, w):
    n = h.shape[0]
    k = int(math.ceil(RATIO * n))
    s = jnp.tanh((h @ w) / jnp.linalg.norm(w))
    sv = jnp.sort(s)
    thresh = sv[n - k]
    gt = s > thresh
    cnt_gt = jnp.sum(gt.astype(jnp.int32))
    eq = s == thresh
    tie_ok = jnp.cumsum(eq.astype(jnp.int32)) <= (k - cnt_gt)
    mask = gt | (eq & tie_ok)
    csum = jnp.cumsum(mask.astype(jnp.int32))
    newidx = jnp.where(mask, csum - 1, -1).astype(jnp.int32)
    sel = jnp.flatnonzero(mask, size=k)
    vals = s[sel]
    h_new = h[sel] * vals[:, None]
    ns, nd = _make_remap(n)(src, dst, newidx)
    return h_new, ns, nd


def _readout(h):
    return jnp.concatenate(
        [jnp.max(h, axis=0, keepdims=True), jnp.mean(h, axis=0, keepdims=True)],
        axis=1)


def _mlp_body(embed_ref, w1_ref, b1_ref, w2_ref, b2_ref, w3_ref, b3_ref,
              out_ref, emb_ref):
    e = embed_ref[...]
    t = jnp.maximum(
        jnp.dot(e, w1_ref[...].T, preferred_element_type=jnp.float32)
        + b1_ref[...], 0.0)
    t = jnp.maximum(
        jnp.dot(t, w2_ref[...].T, preferred_element_type=jnp.float32)
        + b2_ref[...], 0.0)
    o = jax.nn.sigmoid(
        jnp.dot(t, w3_ref[...].T, preferred_element_type=jnp.float32)
        + b3_ref[...])
    out_ref[...] = o
    emb_ref[...] = e


def _mlp(embed, lin1_w, lin1_b, lin2_w, lin2_b, lin3_w, lin3_b):
    w3p = jnp.pad(lin3_w, ((0, 7), (0, 0)))
    b3p = jnp.pad(lin3_b.reshape(1, 1), ((0, 0), (0, 7)))
    out2, emb2 = pl.pallas_call(
        _mlp_body,
        out_shape=(
            jax.ShapeDtypeStruct((1, 8), jnp.float32),
            jax.ShapeDtypeStruct((1, 64), jnp.float32),
        ),
    )(embed, lin1_w, lin1_b.reshape(1, -1), lin2_w, lin2_b.reshape(1, -1),
      w3p, b3p)
    return out2[:1, 0], emb2[0]


def kernel(x, edge_index, batch, emb, c1_lin_w, c1_lin_b, c1_upd_w, p1_w,
           c2_lin_w, c2_lin_b, c2_upd_w, p2_w, c3_lin_w, c3_lin_b, c3_upd_w,
           p3_w, lin1_w, lin1_b, lin2_w, lin2_b, lin3_w, lin3_b):
    h = emb[x[:, 0]]
    src = edge_index[0]
    dst = edge_index[1]

    h = _sage(h, src, dst, c1_lin_w, c1_lin_b, c1_upd_w)
    h, src, dst = _pool(h, src, dst, p1_w)
    x1 = _readout(h)
    h = _sage(h, src, dst, c2_lin_w, c2_lin_b, c2_upd_w)
    h, src, dst = _pool(h, src, dst, p2_w)
    x2 = _readout(h)
    h = _sage(h, src, dst, c3_lin_w, c3_lin_b, c3_upd_w)
    h, src, dst = _pool(h, src, dst, p3_w)
    x3 = _readout(h)

    embed = x1 + x2 + x3
    return _mlp(embed, lin1_w, lin1_b, lin2_w, lin2_b, lin3_w, lin3_b)
